# baseline (speedup 1.0000x reference)
import jax
import jax.numpy as jnp
from jax import lax
from jax.experimental import pallas as pl
from jax.experimental.pallas import tpu as pltpu
from jax.experimental.pallas import tpu_sc as plsc

_RES = 512
_FEAT = 48
_NPTS = 1048576

_NC = 2
_NS = 16
_NW = _NC * _NS
_PPW = _NPTS // _NW
_C = 256
_NCHUNK = _PPW // _C
_SLAB = 128
_NSLAB = _C // _SLAB


def _body(xs, ys, tab, out, x_v, y_v, i00, i01, i10, i11,
          w00, w01, w10, w11, r00, r01, r10, r11, o_v, sem):
    cid = lax.axis_index("c")
    sid = lax.axis_index("s")
    wid = sid * _NC + cid

    def chunk(ci, carry):
        base = wid * _PPW + ci * _C
        pltpu.sync_copy(xs.at[pl.ds(base, _C)], x_v)
        pltpu.sync_copy(ys.at[pl.ds(base, _C)], y_v)

        def grp(g, carry2):
            off = g * 16
            sl = pl.ds(off, 16)
            x = x_v[sl] * float(_RES - 1)
            xi = x.astype(jnp.int32)
            xi = jnp.minimum(jnp.maximum(xi, 0), _RES - 2)
            wx = x - xi.astype(jnp.float32)
            y = y_v[sl] * float(_RES - 1)
            yi = y.astype(jnp.int32)
            yi = jnp.minimum(jnp.maximum(yi, 0), _RES - 2)
            wy = y - yi.astype(jnp.float32)
            b = xi * _RES + yi
            i00[sl] = b
            i01[sl] = b + 1
            i10[sl] = b + _RES
            i11[sl] = b + (_RES + 1)
            u = 1.0 - wx
            v = 1.0 - wy
            w00[sl] = u * v
            w01[sl] = u * wy
            w10[sl] = wx * v
            w11[sl] = wx * wy
            return carry2

        lax.fori_loop(0, _C // 16, grp, None)

        copies = []
        for iv, rv in ((i00, r00), (i01, r01), (i10, r10), (i11, r11)):
            for s in range(_NSLAB):
                copies.append(pltpu.async_copy(
                    tab.at[iv.at[pl.ds(s * _SLAB, _SLAB)]],
                    rv.at[pl.ds(s * _SLAB, _SLAB)], sem))
        for cp in copies:
            cp.wait()

        def pt(g, carry2):
            off = g * 16
            sl = pl.ds(off, 16)
            v00 = w00[sl]
            v01 = w01[sl]
            v10 = w10[sl]
            v11 = w11[sl]
            for k in range(16):
                p = off + k
                s00 = v00[k]
                s01 = v01[k]
                s10 = v10[k]
                s11 = v11[k]
                for j in range(_FEAT // 16):
                    fsl = pl.ds(j * 16, 16)
                    o_v[pl.ds(p * _FEAT + j * 16, 16)] = (
                        r00[p, fsl] * s00 + r01[p, fsl] * s01
                        + r10[p, fsl] * s10 + r11[p, fsl] * s11)
            return carry2

        lax.fori_loop(0, _C // 16, pt, None)
        pltpu.sync_copy(o_v, out.at[pl.ds(base * _FEAT, _C * _FEAT)])
        return carry

    lax.fori_loop(0, _NCHUNK, chunk, None)


def kernel(pts, codebook0):
    xs = pts[:, 0]
    ys = pts[:, 1]
    mesh = plsc.VectorSubcoreMesh(core_axis_name="c", subcore_axis_name="s")
    f = pl.kernel(
        _body,
        mesh=mesh,
        compiler_params=pltpu.CompilerParams(use_tc_tiling_on_sc=False),
        out_type=jax.ShapeDtypeStruct((_NPTS * _FEAT,), jnp.float32),
        scratch_types=[
            pltpu.VMEM((_C,), jnp.float32),
            pltpu.VMEM((_C,), jnp.float32),
            pltpu.VMEM((_C,), jnp.int32),
            pltpu.VMEM((_C,), jnp.int32),
            pltpu.VMEM((_C,), jnp.int32),
            pltpu.VMEM((_C,), jnp.int32),
            pltpu.VMEM((_C,), jnp.float32),
            pltpu.VMEM((_C,), jnp.float32),
            pltpu.VMEM((_C,), jnp.float32),
            pltpu.VMEM((_C,), jnp.float32),
            pltpu.VMEM((_C, _FEAT), jnp.float32),
            pltpu.VMEM((_C, _FEAT), jnp.float32),
            pltpu.VMEM((_C, _FEAT), jnp.float32),
            pltpu.VMEM((_C, _FEAT), jnp.float32),
            pltpu.VMEM((_C * _FEAT,), jnp.float32),
            pltpu.SemaphoreType.DMA,
        ],
    )
    return f(xs, ys, codebook0).reshape(_NPTS, _FEAT)


# speedup vs baseline: 1.2131x; 1.2131x over previous
import jax
import jax.numpy as jnp
from jax import lax
from jax.experimental import pallas as pl
from jax.experimental.pallas import tpu as pltpu
from jax.experimental.pallas import tpu_sc as plsc

_RES = 512
_FEAT = 48
_NPTS = 1048576

_NC = 2
_NS = 16
_NW = _NC * _NS
_PPW = _NPTS // _NW
_C = 128
_NCHUNK = _PPW // _C


def _body(xs, ys, tab, out,
          x0_v, y0_v, i0, w0, r0, o0,
          x1_v, y1_v, i1, w1, r1, o1,
          sg0, sg1, so0, so1):
    cid = lax.axis_index("c")
    sid = lax.axis_index("s")
    wid = sid * _NC + cid
    wbase = wid * _PPW

    bufs = ((x0_v, y0_v, i0, w0, r0, o0, sg0, so0),
            (x1_v, y1_v, i1, w1, r1, o1, sg1, so1))

    def stage_a(c, x_v, y_v, i_v, w_v, r_v, sg):
        base = wbase + c * _C
        pltpu.sync_copy(xs.at[pl.ds(base, _C)], x_v)
        pltpu.sync_copy(ys.at[pl.ds(base, _C)], y_v)

        def grp(g, carry):
            off = g * 16
            sl = pl.ds(off, 16)
            x = x_v[sl] * float(_RES - 1)
            xi = x.astype(jnp.int32)
            xi = jnp.minimum(jnp.maximum(xi, 0), _RES - 2)
            wx = x - xi.astype(jnp.float32)
            y = y_v[sl] * float(_RES - 1)
            yi = y.astype(jnp.int32)
            yi = jnp.minimum(jnp.maximum(yi, 0), _RES - 2)
            wy = y - yi.astype(jnp.float32)
            b = xi * _RES + yi
            i_v[0, sl] = b
            i_v[1, sl] = b + 1
            i_v[2, sl] = b + _RES
            i_v[3, sl] = b + (_RES + 1)
            u = 1.0 - wx
            v = 1.0 - wy
            w_v[0, sl] = u * v
            w_v[1, sl] = u * wy
            w_v[2, sl] = wx * v
            w_v[3, sl] = wx * wy
            return carry

        lax.fori_loop(0, _C // 16, grp, None)
        for t in range(4):
            pltpu.async_copy(tab.at[i_v.at[t]], r_v.at[t], sg)

    def stage_b(c, w_v, r_v, o_v, i_v, sg, so):
        for t in range(4):
            pltpu.make_async_copy(tab.at[i_v.at[t]], r_v.at[t], sg).wait()

        @pl.when(c >= 2)
        def _():
            pltpu.make_async_copy(
                o_v, out.at[pl.ds((wbase + (c - 2) * _C) * _FEAT,
                                  _C * _FEAT)], so).wait()

        def grp(g, carry):
            off = g * 16
            sl = pl.ds(off, 16)
            v00 = w_v[0, sl]
            v01 = w_v[1, sl]
            v10 = w_v[2, sl]
            v11 = w_v[3, sl]
            for k in range(16):
                p = off + k
                s00 = v00[k]
                s01 = v01[k]
                s10 = v10[k]
                s11 = v11[k]
                for j in range(_FEAT // 16):
                    fsl = pl.ds(j * 16, 16)
                    o_v[pl.ds(p * _FEAT + j * 16, 16)] = (
                        r_v[0, p, fsl] * s00 + r_v[1, p, fsl] * s01
                        + r_v[2, p, fsl] * s10 + r_v[3, p, fsl] * s11)
            return carry

        lax.fori_loop(0, _C // 16, grp, None)
        pltpu.async_copy(
            o_v, out.at[pl.ds((wbase + c * _C) * _FEAT, _C * _FEAT)], so)

    stage_a(0, bufs[0][0], bufs[0][1], bufs[0][2], bufs[0][3], bufs[0][4],
            bufs[0][6])

    def pair(ci, carry):
        for u in range(2):
            c = ci * 2 + u
            cur = bufs[u]
            nxt = bufs[1 - u]

            @pl.when(c + 1 < _NCHUNK)
            def _():
                stage_a(c + 1, nxt[0], nxt[1], nxt[2], nxt[3], nxt[4],
                        nxt[6])

            stage_b(c, cur[3], cur[4], cur[5], cur[2], cur[6], cur[7])
        return carry

    lax.fori_loop(0, _NCHUNK // 2, pair, None)

    for u, c in ((0, _NCHUNK - 2), (1, _NCHUNK - 1)):
        pltpu.make_async_copy(
            bufs[u][5],
            out.at[pl.ds((wbase + c * _C) * _FEAT, _C * _FEAT)],
            bufs[u][7]).wait()


def kernel(pts, codebook0):
    xs = pts[:, 0]
    ys = pts[:, 1]
    mesh = plsc.VectorSubcoreMesh(core_axis_name="c", subcore_axis_name="s")
    buf_set = [
        pltpu.VMEM((_C,), jnp.float32),
        pltpu.VMEM((_C,), jnp.float32),
        pltpu.VMEM((4, _C), jnp.int32),
        pltpu.VMEM((4, _C), jnp.float32),
        pltpu.VMEM((4, _C, _FEAT), jnp.float32),
        pltpu.VMEM((_C * _FEAT,), jnp.float32),
    ]
    f = pl.kernel(
        _body,
        mesh=mesh,
        compiler_params=pltpu.CompilerParams(use_tc_tiling_on_sc=False),
        out_type=jax.ShapeDtypeStruct((_NPTS * _FEAT,), jnp.float32),
        scratch_types=buf_set + buf_set + [
            pltpu.SemaphoreType.DMA,
            pltpu.SemaphoreType.DMA,
            pltpu.SemaphoreType.DMA,
            pltpu.SemaphoreType.DMA,
        ],
    )
    return f(xs, ys, codebook0).reshape(_NPTS, _FEAT)
